# baseline (speedup 1.0000x reference)
import functools

import jax
import jax.numpy as jnp
from jax import lax
from jax.experimental import pallas as pl
from jax.experimental.pallas import tpu as pltpu
from jax.experimental.pallas import tpu_sc as plsc

_K = 9
_R = 512

_F32 = jnp.float32


def _pcall(body, **kw):
    return pl.pallas_call(body, **kw)


def _gelu_exact(v):
    return 0.5 * v * (1.0 + lax.erf(v * 0.7071067811865476))


def _dotT(a, b):
    return lax.dot_general(a, b, (((1,), (1,)), ((), ())),
                           preferred_element_type=_F32)


def _gram(a):
    return lax.dot_general(a, a, (((0,), (0,)), ((), ())),
                           preferred_element_type=_F32)


def _topk_stats(x):
    B, N, C = x.shape
    nr = N // _R

    def body(xc_ref, xb_ref, idx_ref, s_ref, g_ref):
        b = pl.program_id(0)
        r = pl.program_id(1)
        step = b * nr + r
        xc = xc_ref[0]
        xb = xb_ref[0]
        sim = _dotT(xc, xb)
        iota = lax.broadcasted_iota(jnp.int32, (_R, N), 1)
        cols = []
        for _ in range(_K):
            idxt = jnp.argmax(sim, axis=1).astype(jnp.int32)
            cols.append(idxt)
            sim = jnp.where(iota == idxt[:, None], -jnp.inf, sim)
        idx_ref[...] = jnp.stack(cols, axis=0)

        @pl.when(step == 0)
        def _():
            s_ref[...] = jnp.zeros_like(s_ref)
            g_ref[...] = jnp.zeros_like(g_ref)

        s_ref[...] += jnp.sum(xc, axis=0, keepdims=True)
        g_ref[...] += _gram(xc)

    return _pcall(
        body,
        grid=(B, nr),
        in_specs=[
            pl.BlockSpec((1, _R, C), lambda b, r: (b, r, 0)),
            pl.BlockSpec((1, N, C), lambda b, r: (b, 0, 0)),
        ],
        out_specs=[
            pl.BlockSpec((_K, _R), lambda b, r: (0, b * nr + r)),
            pl.BlockSpec((1, C), lambda b, r: (0, 0)),
            pl.BlockSpec((C, C), lambda b, r: (0, 0)),
        ],
        out_shape=[
            jax.ShapeDtypeStruct((_K, B * N), jnp.int32),
            jax.ShapeDtypeStruct((1, C), _F32),
            jax.ShapeDtypeStruct((C, C), _F32),
        ],
    )(x, x)


def _bn_fold(S, G, m, p):
    W1, b1, g, be = p["W1"], p["b1"], p["g"], p["be"]
    xbar = S[0] / m
    sig = G / m - jnp.outer(xbar, xbar)
    mu = W1 @ xbar + b1
    var = jnp.sum((W1 @ sig) * W1, axis=1)
    scale = g * lax.rsqrt(var + 1e-5)
    A = W1.T * scale[None, :]
    c = (b1 - mu) * scale + be
    return A, c[None, :]


def _mlp_pass(xin, extra, A, c, W2, b2, post_gelu, with_stats):
    M, C = xin.shape
    H = A.shape[1]
    nsteps = M // _R
    n_extra = 0 if extra is None else 1

    def body(*refs):
        i = 0
        xc_ref = refs[i]; i += 1
        ex_ref = None
        if n_extra:
            ex_ref = refs[i]; i += 1
        A_ref = refs[i]; i += 1
        c_ref = refs[i]; i += 1
        W2_ref = refs[i]; i += 1
        b2_ref = refs[i]; i += 1
        out_ref = refs[i]; i += 1
        if with_stats:
            s_ref = refs[i]; i += 1
            g_ref = refs[i]; i += 1

        xc = xc_ref[...]
        h = jnp.dot(xc, A_ref[...], preferred_element_type=_F32) + c_ref[...]
        h = _gelu_exact(h)
        out = _dotT(h, W2_ref[...]) + b2_ref[...] + xc
        if n_extra:
            out = out + ex_ref[...]
        if post_gelu:
            out = _gelu_exact(out)
        out_ref[...] = out
        if with_stats:
            step = pl.program_id(0)

            @pl.when(step == 0)
            def _():
                s_ref[...] = jnp.zeros_like(s_ref)
                g_ref[...] = jnp.zeros_like(g_ref)

            s_ref[...] += jnp.sum(out, axis=0, keepdims=True)
            g_ref[...] += _gram(out)

    in_specs = [pl.BlockSpec((_R, C), lambda i: (i, 0))]
    ins = [xin]
    if n_extra:
        in_specs.append(pl.BlockSpec((_R, C), lambda i: (i, 0)))
        ins.append(extra)
    in_specs += [
        pl.BlockSpec((C, H), lambda i: (0, 0)),
        pl.BlockSpec((1, H), lambda i: (0, 0)),
        pl.BlockSpec((C, H), lambda i: (0, 0)),
        pl.BlockSpec((1, C), lambda i: (0, 0)),
    ]
    ins += [A, c, W2, b2[None, :]]
    out_specs = [pl.BlockSpec((_R, C), lambda i: (i, 0))]
    out_shape = [jax.ShapeDtypeStruct((M, C), _F32)]
    if with_stats:
        out_specs += [
            pl.BlockSpec((1, C), lambda i: (0, 0)),
            pl.BlockSpec((C, C), lambda i: (0, 0)),
        ]
        out_shape += [
            jax.ShapeDtypeStruct((1, C), _F32),
            jax.ShapeDtypeStruct((C, C), _F32),
        ]
    res = _pcall(
        body,
        grid=(nsteps,),
        in_specs=in_specs,
        out_specs=out_specs,
        out_shape=out_shape,
    )(*ins)
    return res if with_stats else (res[0], None, None)


def _fc_pass(h, mx, Wd, Wo, bfc):
    M, CP = h.shape
    C = Wd.shape[0]
    nsteps = M // _R

    def body(h_ref, m_ref, wd_ref, wo_ref, b_ref, out_ref, s_ref, g_ref):
        hc = h_ref[...][:, :C]
        mc = m_ref[...][:, :C]
        fc = _dotT(hc, wd_ref[...]) + _dotT(mc, wo_ref[...]) + b_ref[...]
        g1 = _gelu_exact(fc)
        out_ref[...] = g1
        step = pl.program_id(0)

        @pl.when(step == 0)
        def _():
            s_ref[...] = jnp.zeros_like(s_ref)
            g_ref[...] = jnp.zeros_like(g_ref)

        s_ref[...] += jnp.sum(g1, axis=0, keepdims=True)
        g_ref[...] += _gram(g1)

    return _pcall(
        body,
        grid=(nsteps,),
        in_specs=[
            pl.BlockSpec((_R, CP), lambda i: (i, 0)),
            pl.BlockSpec((_R, CP), lambda i: (i, 0)),
            pl.BlockSpec((C, C), lambda i: (0, 0)),
            pl.BlockSpec((C, C), lambda i: (0, 0)),
            pl.BlockSpec((1, C), lambda i: (0, 0)),
        ],
        out_specs=[
            pl.BlockSpec((_R, C), lambda i: (i, 0)),
            pl.BlockSpec((1, C), lambda i: (0, 0)),
            pl.BlockSpec((C, C), lambda i: (0, 0)),
        ],
        out_shape=[
            jax.ShapeDtypeStruct((M, C), _F32),
            jax.ShapeDtypeStruct((1, C), _F32),
            jax.ShapeDtypeStruct((C, C), _F32),
        ],
    )(h, mx, Wd, Wo, bfc[None, :])


def _sc_gather_max(h1d, idx9m, C):
    MC = h1d.shape[0]
    M = MC // C
    NW = 32
    rows_w = M // NW
    wsz = rows_w * C
    G16 = rows_w // 16
    GPC = 8
    CH = GPC * 16

    mesh = plsc.VectorSubcoreMesh(core_axis_name="c", subcore_axis_name="s")

    @functools.partial(
        pl.kernel,
        out_type=jax.ShapeDtypeStruct((MC,), _F32),
        mesh=mesh,
        compiler_params=pltpu.CompilerParams(needs_layout_passes=False),
        scratch_types=[
            pltpu.VMEM((wsz,), _F32),
            pltpu.VMEM((_K, rows_w), jnp.int32),
            pltpu.VMEM((CH * C,), _F32),
        ],
    )
    def body(h_hbm, idx_hbm, out_hbm, hv, iv, ov):
        wid = lax.axis_index("c") * 16 + lax.axis_index("s")
        row0 = pl.multiple_of(wid * rows_w, rows_w)
        base0 = pl.multiple_of(wid * wsz, wsz)
        pltpu.sync_copy(h_hbm.at[pl.ds(base0, wsz)], hv)
        pltpu.sync_copy(idx_hbm.at[:, pl.ds(row0, rows_w)], iv)
        lane = lax.broadcasted_iota(jnp.int32, (16,), 0)

        def group(g, carry):
            gbase = pl.multiple_of(g * 16, 16)
            addrs = [iv[t, pl.ds(gbase, 16)] * C for t in range(_K)]
            oaddr = ((g % GPC) * 16 + lane) * C
            for c in range(C):
                acc = plsc.load_gather(hv, [addrs[0] + c])
                for t in range(1, _K):
                    acc = jnp.maximum(acc, plsc.load_gather(hv, [addrs[t] + c]))
                plsc.store_scatter(ov, [oaddr + c], acc)

            @pl.when(g % GPC == GPC - 1)
            def _():
                base = pl.multiple_of(base0 + (g - (GPC - 1)) * 16 * C, CH * C)
                pltpu.sync_copy(ov, out_hbm.at[pl.ds(base, CH * C)])

            return carry

        lax.fori_loop(0, G16, group, 0)

    return body(h1d, idx9m)


def kernel(x, params):
    B, N, C = x.shape
    M = B * N
    xf = x.reshape(M, C)

    gidx, S0, G0 = _topk_stats(x)

    A1, c1 = _bn_fold(S0, G0, M, params["il1"])
    h, _, _ = _mlp_pass(xf, None, A1, c1, params["il1"]["W2"],
                        params["il1"]["b2"], post_gelu=False, with_stats=False)

    mx = _sc_gather_max(h.reshape(M * C), gidx, C).reshape(M, C)

    Wfc, bfc = params["fc"]["W"], params["fc"]["b"]
    We = Wfc[:, 0::2]
    Wo = Wfc[:, 1::2]
    g1, S1, G1 = _fc_pass(h, mx, We - Wo, Wo, bfc)

    A2, c2 = _bn_fold(S1, G1, M, params["ol1"])
    h2, S2, G2 = _mlp_pass(g1, xf, A2, c2, params["ol1"]["W2"],
                           params["ol1"]["b2"], post_gelu=False,
                           with_stats=True)

    A3, c3 = _bn_fold(S2, G2, M, params["il2"])
    g2, S3, G3 = _mlp_pass(h2, None, A3, c3, params["il2"]["W2"],
                           params["il2"]["b2"], post_gelu=True,
                           with_stats=True)

    A4, c4 = _bn_fold(S3, G3, M, params["ol2"])
    out, _, _ = _mlp_pass(g2, h2, A4, c4, params["ol2"]["W2"],
                          params["ol2"]["b2"], post_gelu=False,
                          with_stats=False)
    return out.reshape(B, N, C)


# speedup vs baseline: 1.7510x; 1.7510x over previous
import functools

import jax
import jax.numpy as jnp
from jax import lax
from jax.experimental import pallas as pl
from jax.experimental.pallas import tpu as pltpu
from jax.experimental.pallas import tpu_sc as plsc

_K = 9
_R = 1024

_F32 = jnp.float32


def _pcall(body, **kw):
    return pl.pallas_call(body, **kw)


def _gelu_exact(v):
    return 0.5 * v * (1.0 + lax.erf(v * 0.7071067811865476))


def _dotT(a, b):
    return lax.dot_general(a, b, (((1,), (1,)), ((), ())),
                           preferred_element_type=_F32)


def _gram(a):
    return lax.dot_general(a, a, (((0,), (0,)), ((), ())),
                           preferred_element_type=_F32)


def _topk_stats(x):
    B, N, C = x.shape
    nr = N // _R

    def body(xc_ref, xb_ref, idx_ref, s_ref, g_ref):
        b = pl.program_id(0)
        r = pl.program_id(1)
        step = b * nr + r
        xc = xc_ref[0]
        xb = xb_ref[0]
        sim = _dotT(xc, xb)
        iota = lax.broadcasted_iota(jnp.int32, (_R, N), 1)
        cols = []
        for _ in range(_K):
            idxt = jnp.argmax(sim, axis=1).astype(jnp.int32)
            cols.append(idxt)
            sim = jnp.where(iota == idxt[:, None], -jnp.inf, sim)
        idx_ref[0] = jnp.stack(cols, axis=1) + b * N

        @pl.when(step == 0)
        def _():
            s_ref[...] = jnp.zeros_like(s_ref)
            g_ref[...] = jnp.zeros_like(g_ref)

        s_ref[...] += jnp.sum(xc, axis=0, keepdims=True)
        g_ref[...] += _gram(xc)

    return _pcall(
        body,
        grid=(B, nr),
        in_specs=[
            pl.BlockSpec((1, _R, C), lambda b, r: (b, r, 0)),
            pl.BlockSpec((1, N, C), lambda b, r: (b, 0, 0)),
        ],
        out_specs=[
            pl.BlockSpec((1, _R, _K), lambda b, r: (b, r, 0)),
            pl.BlockSpec((1, C), lambda b, r: (0, 0)),
            pl.BlockSpec((C, C), lambda b, r: (0, 0)),
        ],
        out_shape=[
            jax.ShapeDtypeStruct((B, N, _K), jnp.int32),
            jax.ShapeDtypeStruct((1, C), _F32),
            jax.ShapeDtypeStruct((C, C), _F32),
        ],
    )(x, x)


def _bn_fold(S, G, m, p):
    W1, b1, g, be = p["W1"], p["b1"], p["g"], p["be"]
    xbar = S[0] / m
    sig = G / m - jnp.outer(xbar, xbar)
    mu = W1 @ xbar + b1
    var = jnp.sum((W1 @ sig) * W1, axis=1)
    scale = g * lax.rsqrt(var + 1e-5)
    A = W1.T * scale[None, :]
    c = (b1 - mu) * scale + be
    return A, c[None, :]


def _mlp_pass(xin, extra, A, c, W2, b2, post_gelu, with_stats, pad_to=None):
    M, C = xin.shape
    H = A.shape[1]
    nsteps = M // _R
    n_extra = 0 if extra is None else 1
    CO = pad_to if pad_to is not None else C

    def body(*refs):
        i = 0
        xc_ref = refs[i]; i += 1
        ex_ref = None
        if n_extra:
            ex_ref = refs[i]; i += 1
        A_ref = refs[i]; i += 1
        c_ref = refs[i]; i += 1
        W2_ref = refs[i]; i += 1
        b2_ref = refs[i]; i += 1
        out_ref = refs[i]; i += 1
        if with_stats:
            s_ref = refs[i]; i += 1
            g_ref = refs[i]; i += 1

        xc = xc_ref[...]
        h = jnp.dot(xc, A_ref[...], preferred_element_type=_F32) + c_ref[...]
        h = _gelu_exact(h)
        out = _dotT(h, W2_ref[...]) + b2_ref[...] + xc
        if n_extra:
            out = out + ex_ref[...]
        if post_gelu:
            out = _gelu_exact(out)
        if CO != C:
            out_ref[...] = jnp.concatenate(
                [out, jnp.zeros((out.shape[0], CO - C), _F32)], axis=1)
        else:
            out_ref[...] = out
        if with_stats:
            step = pl.program_id(0)

            @pl.when(step == 0)
            def _():
                s_ref[...] = jnp.zeros_like(s_ref)
                g_ref[...] = jnp.zeros_like(g_ref)

            s_ref[...] += jnp.sum(out, axis=0, keepdims=True)
            g_ref[...] += _gram(out)

    in_specs = [pl.BlockSpec((_R, C), lambda i: (i, 0))]
    ins = [xin]
    if n_extra:
        in_specs.append(pl.BlockSpec((_R, C), lambda i: (i, 0)))
        ins.append(extra)
    in_specs += [
        pl.BlockSpec((C, H), lambda i: (0, 0)),
        pl.BlockSpec((1, H), lambda i: (0, 0)),
        pl.BlockSpec((C, H), lambda i: (0, 0)),
        pl.BlockSpec((1, C), lambda i: (0, 0)),
    ]
    ins += [A, c, W2, b2[None, :]]
    out_specs = [pl.BlockSpec((_R, CO), lambda i: (i, 0))]
    out_shape = [jax.ShapeDtypeStruct((M, CO), _F32)]
    if with_stats:
        out_specs += [
            pl.BlockSpec((1, C), lambda i: (0, 0)),
            pl.BlockSpec((C, C), lambda i: (0, 0)),
        ]
        out_shape += [
            jax.ShapeDtypeStruct((1, C), _F32),
            jax.ShapeDtypeStruct((C, C), _F32),
        ]
    res = _pcall(
        body,
        grid=(nsteps,),
        in_specs=in_specs,
        out_specs=out_specs,
        out_shape=out_shape,
    )(*ins)
    return res if with_stats else (res[0], None, None)


def _fc_pass(h, mx, Wd, Wo, bfc):
    M, CP = h.shape
    C = Wd.shape[0]
    nsteps = M // _R

    def body(h_ref, m_ref, wd_ref, wo_ref, b_ref, out_ref, s_ref, g_ref):
        hc = h_ref[...][:, :C]
        mc = m_ref[...][:, :C]
        fc = _dotT(hc, wd_ref[...]) + _dotT(mc, wo_ref[...]) + b_ref[...]
        g1 = _gelu_exact(fc)
        out_ref[...] = g1
        step = pl.program_id(0)

        @pl.when(step == 0)
        def _():
            s_ref[...] = jnp.zeros_like(s_ref)
            g_ref[...] = jnp.zeros_like(g_ref)

        s_ref[...] += jnp.sum(g1, axis=0, keepdims=True)
        g_ref[...] += _gram(g1)

    return _pcall(
        body,
        grid=(nsteps,),
        in_specs=[
            pl.BlockSpec((_R, CP), lambda i: (i, 0)),
            pl.BlockSpec((_R, CP), lambda i: (i, 0)),
            pl.BlockSpec((C, C), lambda i: (0, 0)),
            pl.BlockSpec((C, C), lambda i: (0, 0)),
            pl.BlockSpec((1, C), lambda i: (0, 0)),
        ],
        out_specs=[
            pl.BlockSpec((_R, C), lambda i: (i, 0)),
            pl.BlockSpec((1, C), lambda i: (0, 0)),
            pl.BlockSpec((C, C), lambda i: (0, 0)),
        ],
        out_shape=[
            jax.ShapeDtypeStruct((M, C), _F32),
            jax.ShapeDtypeStruct((1, C), _F32),
            jax.ShapeDtypeStruct((C, C), _F32),
        ],
    )(h, mx, Wd, Wo, bfc[None, :])


def _sc_gather_max(h_flat, gidx2d, c_used):
    M, C = h_flat.shape
    NW = 32
    rows_w = M // NW
    RPB = 8
    JB = 8
    CH = RPB * JB
    nch = rows_w // CH
    NL = c_used // 16

    mesh = plsc.VectorSubcoreMesh(core_axis_name="c", subcore_axis_name="s")

    @functools.partial(
        pl.kernel,
        out_type=jax.ShapeDtypeStruct((M, C), _F32),
        mesh=mesh,
        scratch_types=[
            pltpu.VMEM((JB, RPB * _K), jnp.int32),
            pltpu.VMEM((JB, RPB * _K, C), _F32),
            pltpu.VMEM((CH, C), _F32),
            pltpu.SemaphoreType.DMA,
        ],
    )
    def body(h_hbm, gidx_hbm, out_hbm, idx_v, gat_v, out_v, sem):
        wid = lax.axis_index("c") * 16 + lax.axis_index("s")

        def chunk(ci, carry):
            row0 = pl.multiple_of(wid * rows_w + ci * CH, CH)
            irow0 = pl.multiple_of(row0 // RPB, JB)
            pltpu.sync_copy(gidx_hbm.at[pl.ds(irow0, JB)], idx_v)
            cps = [
                pltpu.async_copy(h_hbm.at[idx_v.at[j]], gat_v.at[j], sem)
                for j in range(JB)
            ]
            for cp in cps:
                cp.wait()

            def jloop(j, c2):
                for r in range(RPB):
                    for ch in range(NL):
                        sl = pl.ds(ch * 16, 16)
                        acc = gat_v[j, r * _K, sl]
                        for t in range(1, _K):
                            acc = jnp.maximum(acc, gat_v[j, r * _K + t, sl])
                        out_v[j * RPB + r, sl] = acc
                return c2

            lax.fori_loop(0, JB, jloop, 0)
            pltpu.sync_copy(out_v, out_hbm.at[pl.ds(row0, CH)])
            return carry

        lax.fori_loop(0, nch, chunk, 0)

    return body(h_flat, gidx2d)


def kernel(x, params):
    B, N, C = x.shape
    M = B * N
    xf = x.reshape(M, C)

    gidx, S0, G0 = _topk_stats(x)
    gidx2d = gidx.reshape(M * _K // 72, 72)

    A1, c1 = _bn_fold(S0, G0, M, params["il1"])
    h, _, _ = _mlp_pass(xf, None, A1, c1, params["il1"]["W2"],
                        params["il1"]["b2"], post_gelu=False, with_stats=False,
                        pad_to=128)

    mx = _sc_gather_max(h, gidx2d, C)

    Wfc, bfc = params["fc"]["W"], params["fc"]["b"]
    We = Wfc[:, 0::2]
    Wo = Wfc[:, 1::2]
    g1, S1, G1 = _fc_pass(h, mx, We - Wo, Wo, bfc)

    A2, c2 = _bn_fold(S1, G1, M, params["ol1"])
    h2, S2, G2 = _mlp_pass(g1, xf, A2, c2, params["ol1"]["W2"],
                           params["ol1"]["b2"], post_gelu=False,
                           with_stats=True)

    A3, c3 = _bn_fold(S2, G2, M, params["il2"])
    g2, S3, G3 = _mlp_pass(h2, None, A3, c3, params["il2"]["W2"],
                           params["il2"]["b2"], post_gelu=True,
                           with_stats=True)

    A4, c4 = _bn_fold(S3, G3, M, params["ol2"])
    out, _, _ = _mlp_pass(g2, h2, A4, c4, params["ol2"]["W2"],
                          params["ol2"]["b2"], post_gelu=False,
                          with_stats=False)
    return out.reshape(B, N, C)
